# baseline (device time: 15894 ns/iter reference)
import jax
import jax.numpy as jnp
from jax import lax
from jax.experimental import pallas as pl
from jax.experimental.pallas import tpu as pltpu

N_DEV = 8
N_TOK = 1024
D_IN = 256
D_OUT = 512
E_TOTAL = 32
E_LOCAL = E_TOTAL // N_DEV
CAP = 25
TOK_PER = N_TOK // N_DEV
FRAME = 48
GSLOTS = E_LOCAL * 32

BF = jnp.bfloat16
F32 = jnp.float32


def kernel(x, router_W, route_idx, expert_W):
    del router_W

    def body(x_ref, e_ref, w_ref, out_ref, slot_ref, sm_ref, gs_ref,
             sendbuf_ref, recvbuf_ref, send_sems, recv_sems):
        my_pos = lax.axis_index("i")

        barrier = pltpu.get_barrier_semaphore()
        for d in range(N_DEV):
            @pl.when(my_pos != d)
            def _():
                pl.semaphore_signal(
                    barrier, inc=1,
                    device_id=(d,), device_id_type=pl.DeviceIdType.MESH,
                )

        e = e_ref[...]
        oh = (e == lax.broadcasted_iota(jnp.int32, (N_TOK, E_TOTAL), 1))
        ohb = oh.astype(BF)
        ri = lax.broadcasted_iota(jnp.int32, (N_TOK, N_TOK), 0)
        ci = lax.broadcasted_iota(jnp.int32, (N_TOK, N_TOK), 1)
        btri = jnp.logical_and(
            ri > ci, (ri // TOK_PER) == (ci // TOK_PER)).astype(BF)

        wpos = jnp.dot(btri, ohb, preferred_element_type=F32)
        blkT = ((lax.broadcasted_iota(jnp.int32, (N_TOK, N_DEV), 0)
                 // TOK_PER)
                == lax.broadcasted_iota(jnp.int32, (N_TOK, N_DEV), 1))
        blkTb = blkT.astype(BF)
        cnt8 = lax.dot_general(blkTb, ohb, (((0,), (0,)), ((), ())),
                               preferred_element_type=F32)
        tri8 = (lax.broadcasted_iota(jnp.int32, (N_DEV, N_DEV), 0)
                > lax.broadcasted_iota(jnp.int32, (N_DEV, N_DEV), 1))
        off8 = jnp.dot(tri8.astype(BF), cnt8.astype(BF),
                       preferred_element_type=F32)
        offexp = jnp.dot(blkTb, off8.astype(BF),
                         preferred_element_type=F32)
        rank = jnp.sum((wpos + offexp) * oh.astype(F32), axis=1,
                       keepdims=True)
        kept = rank < CAP
        owner = e // E_LOCAL

        dev_iota = lax.broadcasted_iota(jnp.int32, (N_TOK, N_DEV), 1)
        ind_all = jnp.logical_and(kept, owner == dev_iota)
        pr8 = jnp.dot(btri, ind_all.astype(BF),
                      preferred_element_type=F32)
        slotc = jnp.where(ind_all, pr8, -1.0)
        slot_ref[...] = slotc
        my_col = (dev_iota == my_pos).astype(F32)
        sm_ref[...] = jnp.sum(slotc * my_col, axis=1, keepdims=True)

        mine = jnp.logical_and(kept, owner == my_pos)
        gs_ref[...] = jnp.where(
            mine, (e - my_pos * E_LOCAL).astype(F32) * 32.0 + rank, -1.0)

        gsT = (gs_ref[...]
               == lax.broadcasted_iota(jnp.int32, (N_TOK, GSLOTS), 1)
               .astype(F32)).astype(BF)
        xb = x_ref[...].astype(BF)
        gx = lax.dot_general(gsT, xb, (((0,), (0,)), ((), ())),
                             preferred_element_type=F32)
        gxb = gx.astype(BF)
        egrp = (lax.broadcasted_iota(jnp.int32, (GSLOTS, E_LOCAL * D_IN), 1)
                // D_IN
                == lax.broadcasted_iota(jnp.int32, (GSLOTS, E_LOCAL * D_IN),
                                        0) // 32).astype(BF)
        gxcat = jnp.concatenate([gxb] * E_LOCAL, axis=1) * egrp
        wcat = w_ref[...].astype(BF).reshape(E_LOCAL * D_IN, D_OUT)
        gy = jnp.dot(gxcat, wcat, preferred_element_type=F32)
        gyb = gy.astype(BF)

        frame_iota = lax.broadcasted_iota(
            jnp.int32, (TOK_PER, FRAME), 1).astype(F32)
        g_iota = lax.broadcasted_iota(
            jnp.int32, (TOK_PER, GSLOTS), 1).astype(F32)

        for j in range(1, N_DEV):
            k = (my_pos + j) % N_DEV
            smk = sm_ref[pl.ds(k * TOK_PER, TOK_PER), :]
            gsk = gs_ref[pl.ds(k * TOK_PER, TOK_PER), :]
            a_k = (smk == frame_iota).astype(BF)
            b_k = (gsk == g_iota).astype(BF)
            f_k = lax.dot_general(a_k, b_k, (((0,), (0,)), ((), ())),
                                  preferred_element_type=F32)
            packed = jnp.dot(f_k.astype(BF), gyb,
                             preferred_element_type=F32)
            sendbuf_ref[pl.ds(k, 1), :, :] = packed.astype(BF).reshape(
                1, FRAME, D_OUT)
            if j == 1:
                pl.semaphore_wait(barrier, N_DEV - 1)
            rdma = pltpu.make_async_remote_copy(
                src_ref=sendbuf_ref.at[k],
                dst_ref=recvbuf_ref.at[my_pos],
                send_sem=send_sems.at[k],
                recv_sem=recv_sems.at[my_pos],
                device_id=(k,),
                device_id_type=pl.DeviceIdType.MESH,
            )
            rdma.start()

        gs_blk = gs_ref[pl.ds(my_pos * TOK_PER, TOK_PER), :]
        b_my = (gs_blk == g_iota).astype(BF)
        total = jnp.dot(b_my, gyb, preferred_element_type=F32)
        slot_blk = slot_ref[pl.ds(my_pos * TOK_PER, TOK_PER), :]

        for j in range(1, N_DEV):
            d = (my_pos - j) % N_DEV
            recv = pltpu.make_async_remote_copy(
                src_ref=sendbuf_ref.at[0],
                dst_ref=recvbuf_ref.at[d],
                send_sem=send_sems.at[my_pos],
                recv_sem=recv_sems.at[d],
                device_id=(d,),
                device_id_type=pl.DeviceIdType.MESH,
            )
            recv.wait_recv()
            oh_d = (lax.broadcasted_iota(jnp.int32, (TOK_PER, N_DEV), 1)
                    == d).astype(F32)
            sd = jnp.sum(slot_blk * oh_d, axis=1, keepdims=True)
            scat = (sd == frame_iota).astype(BF)
            frame = recvbuf_ref[pl.ds(d, 1), :, :].reshape(FRAME, D_OUT)
            total += jnp.dot(scat, frame, preferred_element_type=F32)
        out_ref[...] = total

        for j in range(1, N_DEV):
            k = (my_pos + j) % N_DEV
            send = pltpu.make_async_remote_copy(
                src_ref=sendbuf_ref.at[k],
                dst_ref=recvbuf_ref.at[my_pos],
                send_sem=send_sems.at[k],
                recv_sem=recv_sems.at[my_pos],
                device_id=(k,),
                device_id_type=pl.DeviceIdType.MESH,
            )
            send.wait_send()

    return pl.pallas_call(
        body,
        out_shape=jax.ShapeDtypeStruct((TOK_PER, D_OUT), jnp.float32),
        in_specs=[
            pl.BlockSpec(memory_space=pltpu.VMEM),
            pl.BlockSpec(memory_space=pltpu.VMEM),
            pl.BlockSpec(memory_space=pltpu.VMEM),
        ],
        out_specs=pl.BlockSpec(memory_space=pltpu.VMEM),
        scratch_shapes=[
            pltpu.VMEM((N_TOK, N_DEV), F32),
            pltpu.VMEM((N_TOK, 1), F32),
            pltpu.VMEM((N_TOK, 1), F32),
            pltpu.VMEM((N_DEV, FRAME, D_OUT), BF),
            pltpu.VMEM((N_DEV, FRAME, D_OUT), BF),
            pltpu.SemaphoreType.DMA((N_DEV,)),
            pltpu.SemaphoreType.DMA((N_DEV,)),
        ],
        compiler_params=pltpu.CompilerParams(collective_id=0),
    )(x, route_idx, expert_W)


# device time: 14669 ns/iter; 1.0835x vs baseline; 1.0835x over previous
import jax
import jax.numpy as jnp
from jax import lax
from jax.experimental import pallas as pl
from jax.experimental.pallas import tpu as pltpu

N_DEV = 8
N_TOK = 1024
D_IN = 256
D_OUT = 512
E_TOTAL = 32
E_LOCAL = E_TOTAL // N_DEV
CAP = 25
TOK_PER = N_TOK // N_DEV
FRAME = 48
GSLOTS = E_LOCAL * 32

BF = jnp.bfloat16
F32 = jnp.float32


def kernel(x, router_W, route_idx, expert_W):
    del router_W

    def body(x_ref, e_ref, w_ref, out_ref, slot_ref, gs_ref,
             sendbuf_ref, recvbuf_ref, send_sems, recv_sems):
        my_pos = lax.axis_index("i")

        barrier = pltpu.get_barrier_semaphore()
        for d in range(N_DEV):
            @pl.when(my_pos != d)
            def _():
                pl.semaphore_signal(
                    barrier, inc=1,
                    device_id=(d,), device_id_type=pl.DeviceIdType.MESH,
                )

        e = e_ref[...]
        oh = (e == lax.broadcasted_iota(jnp.int32, (N_TOK, E_TOTAL), 1))
        ohb = oh.astype(BF)
        tri128 = (lax.broadcasted_iota(jnp.int32, (TOK_PER, TOK_PER), 0)
                  > lax.broadcasted_iota(jnp.int32, (TOK_PER, TOK_PER), 1)
                  ).astype(BF)

        cnt8 = jnp.concatenate(
            [jnp.sum(ohb[b * TOK_PER:(b + 1) * TOK_PER, :], axis=0,
                     keepdims=True, dtype=F32) for b in range(N_DEV)],
            axis=0)
        tri8 = (lax.broadcasted_iota(jnp.int32, (N_DEV, N_DEV), 0)
                > lax.broadcasted_iota(jnp.int32, (N_DEV, N_DEV), 1))
        off8 = jnp.dot(tri8.astype(BF), cnt8.astype(BF),
                       preferred_element_type=F32)
        rank = jnp.concatenate(
            [jnp.sum(
                (jnp.dot(tri128, ohb[b * TOK_PER:(b + 1) * TOK_PER, :],
                         preferred_element_type=F32)
                 + off8[b:b + 1, :])
                * oh[b * TOK_PER:(b + 1) * TOK_PER, :].astype(F32),
                axis=1, keepdims=True) for b in range(N_DEV)],
            axis=0)
        kept = rank < CAP
        owner = e // E_LOCAL

        dev_iota = lax.broadcasted_iota(jnp.int32, (N_TOK, N_DEV), 1)
        ind_all = jnp.logical_and(kept, owner == dev_iota)
        ind_b = ind_all.astype(BF)
        pr8 = jnp.concatenate(
            [jnp.dot(tri128, ind_b[b * TOK_PER:(b + 1) * TOK_PER, :],
                     preferred_element_type=F32) for b in range(N_DEV)],
            axis=0)
        slotc = jnp.where(ind_all, pr8, -1.0)
        slot_ref[...] = slotc
        my_col = (dev_iota == my_pos).astype(F32)
        sm = jnp.sum(slotc * my_col, axis=1, keepdims=True)

        mine = jnp.logical_and(kept, owner == my_pos)
        gs = jnp.where(
            mine, (e - my_pos * E_LOCAL).astype(F32) * 32.0 + rank, -1.0)
        gs_ref[...] = gs

        gsT = (gs == lax.broadcasted_iota(jnp.int32, (N_TOK, GSLOTS), 1)
               .astype(F32)).astype(BF)
        xb = x_ref[...].astype(BF)
        gx = lax.dot_general(gsT, xb, (((0,), (0,)), ((), ())),
                             preferred_element_type=F32)
        gxb = gx.astype(BF)
        egrp = (lax.broadcasted_iota(jnp.int32, (GSLOTS, E_LOCAL * D_IN), 1)
                // D_IN
                == lax.broadcasted_iota(jnp.int32, (GSLOTS, E_LOCAL * D_IN),
                                        0) // 32).astype(BF)
        gxcat = jnp.concatenate([gxb] * E_LOCAL, axis=1) * egrp
        wcat = w_ref[...].astype(BF).reshape(E_LOCAL * D_IN, D_OUT)
        gy = jnp.dot(gxcat, wcat, preferred_element_type=F32)
        gyb = gy.astype(BF)

        frame_iota = lax.broadcasted_iota(
            jnp.int32, (TOK_PER, FRAME), 1).astype(F32)
        g_iota = lax.broadcasted_iota(
            jnp.int32, (TOK_PER, GSLOTS), 1).astype(F32)

        a_all = (sm == lax.broadcasted_iota(jnp.int32, (N_TOK, FRAME), 1)
                 .astype(F32)).astype(BF)
        f_all = jnp.concatenate(
            [lax.dot_general(
                a_all[b * TOK_PER:(b + 1) * TOK_PER, :],
                gsT[b * TOK_PER:(b + 1) * TOK_PER, :],
                (((0,), (0,)), ((), ())),
                preferred_element_type=F32) for b in range(N_DEV)],
            axis=0)
        frames = jnp.dot(f_all.astype(BF), gyb,
                         preferred_element_type=F32)
        sendbuf_ref[...] = frames.astype(BF).reshape(N_DEV, FRAME, D_OUT)

        pl.semaphore_wait(barrier, N_DEV - 1)
        for j in range(1, N_DEV):
            k = (my_pos + j) % N_DEV
            rdma = pltpu.make_async_remote_copy(
                src_ref=sendbuf_ref.at[k],
                dst_ref=recvbuf_ref.at[my_pos],
                send_sem=send_sems.at[k],
                recv_sem=recv_sems.at[my_pos],
                device_id=(k,),
                device_id_type=pl.DeviceIdType.MESH,
            )
            rdma.start()

        gs_blk = gs_ref[pl.ds(my_pos * TOK_PER, TOK_PER), :]
        b_my = (gs_blk == g_iota).astype(BF)
        total = jnp.dot(b_my, gyb, preferred_element_type=F32)
        slot_blk = slot_ref[pl.ds(my_pos * TOK_PER, TOK_PER), :]

        for j in range(1, N_DEV):
            d = (my_pos - j) % N_DEV
            recv = pltpu.make_async_remote_copy(
                src_ref=sendbuf_ref.at[0],
                dst_ref=recvbuf_ref.at[d],
                send_sem=send_sems.at[my_pos],
                recv_sem=recv_sems.at[d],
                device_id=(d,),
                device_id_type=pl.DeviceIdType.MESH,
            )
            recv.wait_recv()
            oh_d = (lax.broadcasted_iota(jnp.int32, (TOK_PER, N_DEV), 1)
                    == d).astype(F32)
            sd = jnp.sum(slot_blk * oh_d, axis=1, keepdims=True)
            scat = (sd == frame_iota).astype(BF)
            frame = recvbuf_ref[pl.ds(d, 1), :, :].reshape(FRAME, D_OUT)
            total += jnp.dot(scat, frame, preferred_element_type=F32)
        out_ref[...] = total

        for j in range(1, N_DEV):
            k = (my_pos + j) % N_DEV
            send = pltpu.make_async_remote_copy(
                src_ref=sendbuf_ref.at[k],
                dst_ref=recvbuf_ref.at[my_pos],
                send_sem=send_sems.at[k],
                recv_sem=recv_sems.at[my_pos],
                device_id=(k,),
                device_id_type=pl.DeviceIdType.MESH,
            )
            send.wait_send()

    return pl.pallas_call(
        body,
        out_shape=jax.ShapeDtypeStruct((TOK_PER, D_OUT), jnp.float32),
        in_specs=[
            pl.BlockSpec(memory_space=pltpu.VMEM),
            pl.BlockSpec(memory_space=pltpu.VMEM),
            pl.BlockSpec(memory_space=pltpu.VMEM),
        ],
        out_specs=pl.BlockSpec(memory_space=pltpu.VMEM),
        scratch_shapes=[
            pltpu.VMEM((N_TOK, N_DEV), F32),
            pltpu.VMEM((N_TOK, 1), F32),
            pltpu.VMEM((N_DEV, FRAME, D_OUT), BF),
            pltpu.VMEM((N_DEV, FRAME, D_OUT), BF),
            pltpu.SemaphoreType.DMA((N_DEV,)),
            pltpu.SemaphoreType.DMA((N_DEV,)),
        ],
        compiler_params=pltpu.CompilerParams(collective_id=0),
    )(x, route_idx, expert_W)
